# Initial kernel scaffold; baseline (speedup 1.0000x reference)
import jax
import jax.numpy as jnp
from jax.experimental import pallas as pl


def kernel(x, edge_index, perm, W_gcn, b_gcn, prelu_a, W_disc):
    raise NotImplementedError("write your pallas kernel here")


# speedup vs baseline: 24.2950x; 24.2950x over previous
import functools

import jax
import jax.numpy as jnp
from jax import lax
from jax.experimental import pallas as pl
from jax.experimental.pallas import tpu as pltpu
from jax.experimental.pallas import tpu_sc as plsc

N = 10000
E = 320000
D_IN = 128
D_HID = 512

NT = 16
B = 128
EPT = E // NT
NBF = EPT // B
REM = EPT - NBF * B
NRB = N // B
LAST = N - NRB * B
RB_ROUNDS = (NRB + NT - 1) // NT
LAST_TILE = NRB % NT

_F32 = jnp.float32


def _fill1d(ref, n, val):
    vec16 = jnp.full((16,), val, _F32)
    for k in range(n // 16):
        ref[pl.ds(k * 16, 16)] = vec16


def _fill2d(ref, nrows, ncols, val):
    vec16 = jnp.full((16,), val, _F32)
    for r in range(nrows):
        for k in range(ncols // 16):
            ref[r, pl.ds(k * 16, 16)] = vec16


def _sc_mesh():
    return plsc.VectorSubcoreMesh(core_axis_name="c", subcore_axis_name="s")


@functools.partial(
    pl.kernel,
    out_type=(
        jax.ShapeDtypeStruct((N,), _F32),
        jax.ShapeDtypeStruct((N, D_IN), _F32),
    ),
    mesh=_sc_mesh(),
    scratch_types=[
        pltpu.VMEM((B,), jnp.int32),
        pltpu.VMEM((16,), jnp.int32),
        pltpu.VMEM((REM,), jnp.int32),
        pltpu.VMEM((B,), _F32),
        pltpu.VMEM((REM,), _F32),
        pltpu.VMEM((B,), _F32),
        pltpu.VMEM((B, D_IN), _F32),
        pltpu.VMEM((16, D_IN), _F32),
        pltpu.VMEM_SHARED((N,), _F32),
        pltpu.SemaphoreType.DMA,
    ],
)
def _prep(x_hbm, perm_hbm, dst_hbm, deg_hbm, xp_hbm,
          idxb, idxr16, idxr32, ones, onesr, tmp, rows, rowsr, degsh, sem):
    c = lax.axis_index("c")
    s = lax.axis_index("s")

    _fill1d(ones, B, 1.0)
    _fill1d(onesr, REM, 1.0)
    _fill1d(tmp, B, 0.0)

    for k in range(RB_ROUNDS):
        b = s + NT * k

        @pl.when(b < NRB)
        def _():
            pltpu.sync_copy(tmp, degsh.at[pl.ds(b * B, B)])

    @pl.when(s == LAST_TILE)
    def _():
        pltpu.sync_copy(tmp.at[pl.ds(0, LAST)], degsh.at[pl.ds(NRB * B, LAST)])

    plsc.subcore_barrier()

    @pl.when(c == 0)
    def _():
        def body(j, carry):
            base = pl.multiple_of(s * EPT + j * B, 8)
            pltpu.sync_copy(dst_hbm.at[pl.ds(base, B)], idxb)
            pltpu.sync_copy(ones, degsh.at[idxb], add=True)
            return carry

        lax.fori_loop(0, NBF, body, 0)
        base_r = pl.multiple_of(s * EPT + NBF * B, 8)
        pltpu.sync_copy(dst_hbm.at[pl.ds(base_r, REM)], idxr32)
        pltpu.sync_copy(onesr, degsh.at[idxr32], add=True)

    @pl.when(c == 1)
    def _():
        for k in range(RB_ROUNDS):
            b = s + NT * k

            @pl.when(b < NRB)
            def _():
                r0 = pl.multiple_of(b * B, 8)
                pltpu.sync_copy(perm_hbm.at[pl.ds(r0, B)], idxb)
                pltpu.async_copy(x_hbm.at[idxb], rows, sem).wait()
                pltpu.sync_copy(rows, xp_hbm.at[pl.ds(r0, B)])

        @pl.when(s == LAST_TILE)
        def _():
            r0 = NRB * B
            pltpu.sync_copy(perm_hbm.at[pl.ds(r0, LAST)], idxr16)
            pltpu.async_copy(x_hbm.at[idxr16], rowsr, sem).wait()
            pltpu.sync_copy(rowsr, xp_hbm.at[pl.ds(r0, LAST)])

    plsc.subcore_barrier()

    @pl.when(c == 0)
    def _():
        for k in range(RB_ROUNDS):
            b = s + NT * k

            @pl.when(b < NRB)
            def _():
                r0 = pl.multiple_of(b * B, 8)
                pltpu.sync_copy(degsh.at[pl.ds(r0, B)], tmp)
                pltpu.sync_copy(tmp, deg_hbm.at[pl.ds(r0, B)])

        @pl.when(s == LAST_TILE)
        def _():
            r0 = NRB * B
            pltpu.sync_copy(degsh.at[pl.ds(r0, LAST)], tmp.at[pl.ds(0, LAST)])
            pltpu.sync_copy(tmp.at[pl.ds(0, LAST)], deg_hbm.at[pl.ds(r0, LAST)])


@functools.partial(
    pl.kernel,
    out_type=(
        jax.ShapeDtypeStruct((N, D_IN), _F32),
        jax.ShapeDtypeStruct((N, D_IN), _F32),
    ),
    mesh=_sc_mesh(),
    scratch_types=[
        pltpu.VMEM((B,), jnp.int32),
        pltpu.VMEM((B,), jnp.int32),
        pltpu.VMEM((REM,), jnp.int32),
        pltpu.VMEM((REM,), jnp.int32),
        pltpu.VMEM((B, D_IN), _F32),
        pltpu.VMEM((REM, D_IN), _F32),
        pltpu.VMEM_SHARED((N, D_IN), _F32),
        pltpu.SemaphoreType.DMA,
    ],
)
def _agg(y_hbm, y2_hbm, src_hbm, dst_hbm, acc1_hbm, acc2_hbm,
         sidx, didx, sidxr, didxr, rows, rowsr, accsh, sem):
    c = lax.axis_index("c")
    s = lax.axis_index("s")

    _fill2d(rowsr, REM, D_IN, 0.0)
    nz = N // REM
    nz_rounds = (nz + NT - 1) // NT

    def zbody(j, carry):
        m = s + NT * j

        @pl.when(m < nz)
        def _():
            pltpu.sync_copy(rowsr, accsh.at[pl.ds(m * REM, REM)])

        return carry

    lax.fori_loop(0, nz_rounds, zbody, 0)

    @pl.when(s == 0)
    def _():
        pltpu.sync_copy(rowsr.at[pl.ds(0, LAST)], accsh.at[pl.ds(nz * REM, LAST)])

    plsc.subcore_barrier()

    def edge_phase(tbl_hbm):
        def body(j, carry):
            base = pl.multiple_of(s * EPT + j * B, 8)
            pltpu.sync_copy(src_hbm.at[pl.ds(base, B)], sidx)
            pltpu.sync_copy(dst_hbm.at[pl.ds(base, B)], didx)
            pltpu.async_copy(tbl_hbm.at[sidx], rows, sem).wait()
            pltpu.sync_copy(rows, accsh.at[didx], add=True)
            return carry

        lax.fori_loop(0, NBF, body, 0)
        base_r = pl.multiple_of(s * EPT + NBF * B, 8)
        pltpu.sync_copy(src_hbm.at[pl.ds(base_r, REM)], sidxr)
        pltpu.sync_copy(dst_hbm.at[pl.ds(base_r, REM)], didxr)
        pltpu.async_copy(tbl_hbm.at[sidxr], rowsr, sem).wait()
        pltpu.sync_copy(rowsr, accsh.at[didxr], add=True)

    @pl.when(c == 0)
    def _():
        edge_phase(y_hbm)

    @pl.when(c == 1)
    def _():
        edge_phase(y2_hbm)

    plsc.subcore_barrier()

    def drain_phase(out_hbm):
        for k in range(RB_ROUNDS):
            b = s + NT * k

            @pl.when(b < NRB)
            def _():
                r0 = pl.multiple_of(b * B, 8)
                pltpu.sync_copy(accsh.at[pl.ds(r0, B)], rows)
                pltpu.sync_copy(rows, out_hbm.at[pl.ds(r0, B)])

        @pl.when(s == LAST_TILE)
        def _():
            r0 = NRB * B
            pltpu.sync_copy(accsh.at[pl.ds(r0, LAST)], rows.at[pl.ds(0, LAST)])
            pltpu.sync_copy(rows.at[pl.ds(0, LAST)], out_hbm.at[pl.ds(r0, LAST)])

    @pl.when(c == 0)
    def _():
        drain_phase(acc1_hbm)

    @pl.when(c == 1)
    def _():
        drain_phase(acc2_hbm)


_BR = 2000


def _scale_body(deg_ref, x_ref, xp_ref, y_ref, y2_ref, dinv_ref):
    dv = lax.rsqrt(deg_ref[...] + 1.0)
    y_ref[...] = dv * x_ref[...]
    y2_ref[...] = dv * xp_ref[...]
    dinv_ref[...] = dv


def _scale(deg2d, x, xp):
    grid = N // _BR
    return pl.pallas_call(
        _scale_body,
        grid=(grid,),
        in_specs=[
            pl.BlockSpec((_BR, 1), lambda i: (i, 0)),
            pl.BlockSpec((_BR, D_IN), lambda i: (i, 0)),
            pl.BlockSpec((_BR, D_IN), lambda i: (i, 0)),
        ],
        out_specs=[
            pl.BlockSpec((_BR, D_IN), lambda i: (i, 0)),
            pl.BlockSpec((_BR, D_IN), lambda i: (i, 0)),
            pl.BlockSpec((_BR, 1), lambda i: (i, 0)),
        ],
        out_shape=[
            jax.ShapeDtypeStruct((N, D_IN), _F32),
            jax.ShapeDtypeStruct((N, D_IN), _F32),
            jax.ShapeDtypeStruct((N, 1), _F32),
        ],
    )(deg2d, x, xp)


_BD = 1000


def _dense_body(acc1_ref, y_ref, acc2_ref, y2_ref, dinv_ref, w_ref, b_ref,
                a_ref, z1_ref, z2_ref, s1_ref):
    i = pl.program_id(0)
    dv = dinv_ref[...]
    a = a_ref[...]
    w = w_ref[...]
    bb = b_ref[...]
    agg1 = dv * (acc1_ref[...] + y_ref[...])
    u1 = jnp.dot(agg1, w, preferred_element_type=_F32) + bb
    z1 = jnp.where(u1 >= 0.0, u1, a * u1)
    z1_ref[...] = z1
    agg2 = dv * (acc2_ref[...] + y2_ref[...])
    u2 = jnp.dot(agg2, w, preferred_element_type=_F32) + bb
    z2_ref[...] = jnp.where(u2 >= 0.0, u2, a * u2)
    cs = jnp.sum(z1, axis=0, keepdims=True)

    @pl.when(i == 0)
    def _():
        s1_ref[...] = cs

    @pl.when(i > 0)
    def _():
        s1_ref[...] += cs


def _dense(acc1, y, acc2, y2, dinv2d, W, b2d, a2d):
    grid = N // _BD
    return pl.pallas_call(
        _dense_body,
        grid=(grid,),
        in_specs=[
            pl.BlockSpec((_BD, D_IN), lambda i: (i, 0)),
            pl.BlockSpec((_BD, D_IN), lambda i: (i, 0)),
            pl.BlockSpec((_BD, D_IN), lambda i: (i, 0)),
            pl.BlockSpec((_BD, D_IN), lambda i: (i, 0)),
            pl.BlockSpec((_BD, 1), lambda i: (i, 0)),
            pl.BlockSpec((D_IN, D_HID), lambda i: (0, 0)),
            pl.BlockSpec((1, D_HID), lambda i: (0, 0)),
            pl.BlockSpec((1, 1), lambda i: (0, 0)),
        ],
        out_specs=[
            pl.BlockSpec((_BD, D_HID), lambda i: (i, 0)),
            pl.BlockSpec((_BD, D_HID), lambda i: (i, 0)),
            pl.BlockSpec((1, D_HID), lambda i: (0, 0)),
        ],
        out_shape=[
            jax.ShapeDtypeStruct((N, D_HID), _F32),
            jax.ShapeDtypeStruct((N, D_HID), _F32),
            jax.ShapeDtypeStruct((1, D_HID), _F32),
        ],
    )(acc1, y, acc2, y2, dinv2d, W, b2d, a2d)


def _disc_body(s1_ref, wd_ref, z1_ref, z2_ref, pos_ref, neg_ref, ws_ref):
    i = pl.program_id(0)

    @pl.when(i == 0)
    def _():
        summ = jax.nn.sigmoid(s1_ref[...] * (1.0 / N))
        ws_ref[...] = jnp.sum(wd_ref[...] * summ, axis=1, keepdims=True)

    ws = ws_ref[...]
    pos_ref[...] = jnp.dot(z1_ref[...], ws, preferred_element_type=_F32)
    neg_ref[...] = jnp.dot(z2_ref[...], ws, preferred_element_type=_F32)


def _disc(s1, W_disc, z1, z2):
    grid = N // _BD
    return pl.pallas_call(
        _disc_body,
        grid=(grid,),
        in_specs=[
            pl.BlockSpec((1, D_HID), lambda i: (0, 0)),
            pl.BlockSpec((D_HID, D_HID), lambda i: (0, 0)),
            pl.BlockSpec((_BD, D_HID), lambda i: (i, 0)),
            pl.BlockSpec((_BD, D_HID), lambda i: (i, 0)),
        ],
        out_specs=[
            pl.BlockSpec((_BD, 1), lambda i: (i, 0)),
            pl.BlockSpec((_BD, 1), lambda i: (i, 0)),
        ],
        out_shape=[
            jax.ShapeDtypeStruct((N, 1), _F32),
            jax.ShapeDtypeStruct((N, 1), _F32),
        ],
        scratch_shapes=[pltpu.VMEM((D_HID, 1), _F32)],
    )(s1, W_disc, z1, z2)


def kernel(x, edge_index, perm, W_gcn, b_gcn, prelu_a, W_disc):
    src = edge_index[0]
    dst = edge_index[1]
    degE, xp = _prep(x, perm, dst)
    y, y2, dinv2d = _scale(degE.reshape(N, 1), x, xp)
    acc1, acc2 = _agg(y, y2, src, dst)
    z1, z2, s1 = _dense(acc1, y, acc2, y2, dinv2d, W_gcn,
                        b_gcn.reshape(1, D_HID), prelu_a.reshape(1, 1))
    pos2d, neg2d = _disc(s1, W_disc, z1, z2)
    return (jnp.squeeze(pos2d, -1), jnp.squeeze(neg2d, -1))
